# Initial kernel scaffold; baseline (speedup 1.0000x reference)
import jax
import jax.numpy as jnp
from jax.experimental import pallas as pl


def kernel(input_ids, token_type_ids, word_emb, pos_emb, type_emb, ln_gamma, ln_beta):
    raise NotImplementedError("write your pallas kernel here")


# speedup vs baseline: 2.9951x; 2.9951x over previous
import jax
import jax.numpy as jnp
from jax import lax
from jax.experimental import pallas as pl
from jax.experimental.pallas import tpu as pltpu
from jax.experimental.pallas import tpu_sc as plsc

B, L = 1024, 200
VOCAB, H = 100000, 128
EPS = 1e-12

NC, NS = 2, 16
NW = NC * NS
SEQ_PER_W = B // NW
NF = H // 16


def _sc_body(ids_hbm, tt_hbm, word_hbm, pos_hbm, type_hbm, gam_hbm, bet_hbm,
             out_hbm, idx_a, idx_b, tt_v, rows_v, pos_v, type_v, gam_v, bet_v, sem):
    wid = lax.axis_index("s") * NC + lax.axis_index("c")

    pltpu.sync_copy(pos_hbm.at[pl.ds(0, L)], pos_v)
    pltpu.sync_copy(type_hbm, type_v)
    pltpu.sync_copy(gam_hbm, gam_v)
    pltpu.sync_copy(bet_hbm, bet_v)

    gam = [gam_v[pl.ds(f * 16, 16)] for f in range(NF)]
    bet = [bet_v[pl.ds(f * 16, 16)] for f in range(NF)]

    def do_seq(s, carry):
        seq = wid * SEQ_PER_W + s
        pltpu.sync_copy(ids_hbm.at[seq, 0], idx_a)
        pltpu.sync_copy(ids_hbm.at[seq, 1], idx_b)
        pltpu.sync_copy(tt_hbm.at[seq], tt_v.at[pl.ds(0, L)])
        d0 = pltpu.async_copy(word_hbm.at[idx_a], rows_v.at[pl.ds(0, 100)], sem)
        d1 = pltpu.async_copy(word_hbm.at[idx_b], rows_v.at[pl.ds(100, 100)], sem)
        d0.wait()
        d1.wait()

        def do_tok(t, c2):
            tt = tt_v[pl.ds(t, 16)][0]
            x = []
            for f in range(NF):
                w = rows_v[t, pl.ds(f * 16, 16)]
                p = pos_v[t, pl.ds(f * 16, 16)]
                ty = type_v[tt, pl.ds(f * 16, 16)]
                x.append(w + p + ty)
            acc = x[0]
            for f in range(1, NF):
                acc = acc + x[f]
            mu = jnp.sum(acc) * (1.0 / H)
            c = [xf - mu for xf in x]
            sq = c[0] * c[0]
            for f in range(1, NF):
                sq = sq + c[f] * c[f]
            var = jnp.sum(sq) * (1.0 / H)
            v = jnp.full((16,), var + EPS, dtype=jnp.float32)
            i = lax.bitcast_convert_type(v, jnp.int32)
            i = jnp.int32(0x5F3759DF) - lax.shift_right_arithmetic(i, 1)
            y = lax.bitcast_convert_type(i, jnp.float32)
            for _ in range(3):
                y = y * (1.5 - 0.5 * v * y * y)
            for f in range(NF):
                rows_v[t, pl.ds(f * 16, 16)] = c[f] * y * gam[f] + bet[f]
            return c2

        lax.fori_loop(0, L, do_tok, 0)
        pltpu.sync_copy(rows_v, out_hbm.at[pl.ds(seq * L, L)])
        return carry

    lax.fori_loop(0, SEQ_PER_W, do_seq, 0)


@jax.jit
def _run(ids2, tt1, word_emb, pos_emb, type_emb, ln_gamma, ln_beta):
    mesh = plsc.VectorSubcoreMesh(core_axis_name="c", subcore_axis_name="s",
                                  num_cores=NC, num_subcores=NS)
    k = pl.kernel(
        _sc_body,
        out_type=jax.ShapeDtypeStruct((B * L, H), jnp.float32),
        mesh=mesh,
        scratch_types=[
            pltpu.VMEM((100,), jnp.int32),
            pltpu.VMEM((100,), jnp.int32),
            pltpu.VMEM((L + 16,), jnp.int32),
            pltpu.VMEM((L, H), jnp.float32),
            pltpu.VMEM((L, H), jnp.float32),
            pltpu.VMEM((2, H), jnp.float32),
            pltpu.VMEM((H,), jnp.float32),
            pltpu.VMEM((H,), jnp.float32),
            pltpu.SemaphoreType.DMA,
        ],
        compiler_params=pltpu.CompilerParams(needs_layout_passes=False,
                                             use_tc_tiling_on_sc=False),
    )
    return k(ids2, tt1, word_emb, pos_emb, type_emb, ln_gamma, ln_beta)


def kernel(input_ids, token_type_ids, word_emb, pos_emb, type_emb, ln_gamma, ln_beta):
    ids2 = input_ids.astype(jnp.int32).reshape(B, 2, 100)
    tt1 = token_type_ids.astype(jnp.int32)
    out = _run(ids2, tt1, word_emb, pos_emb, type_emb, ln_gamma, ln_beta)
    return out.reshape(B, L, H)


# speedup vs baseline: 6.9675x; 2.3263x over previous
import jax
import jax.numpy as jnp
from jax import lax
from jax.experimental import pallas as pl
from jax.experimental.pallas import tpu as pltpu
from jax.experimental.pallas import tpu_sc as plsc

B, L = 1024, 200
VOCAB, H = 100000, 128
EPS = 1e-12

NC, NS = 2, 16
NW = NC * NS
SEQ_PER_W = B // NW
CH = 100
NCHUNK = SEQ_PER_W * L // CH
NBUF = 4
GRP = 10
NF = H // 16


def _sc_body(ids_hbm, tt_hbm, word_hbm, pos_hbm, type_hbm, gam_hbm, bet_hbm,
             out_hbm, ids_v, tt_v, pos_v, type_v, gb_v,
             rows0, rows1, rows2, rows3,
             gsem0, gsem1, gsem2, gsem3, ssem0, ssem1, ssem2, ssem3):
    rows = [rows0, rows1, rows2, rows3]
    gsem = [gsem0, gsem1, gsem2, gsem3]
    ssem = [ssem0, ssem1, ssem2, ssem3]

    wid = lax.axis_index("s") * NC + lax.axis_index("c")
    cbase = wid * NCHUNK

    pltpu.sync_copy(ids_hbm.at[pl.ds(wid * NCHUNK, NCHUNK)], ids_v)
    pltpu.sync_copy(tt_hbm.at[pl.ds(wid * NCHUNK * CH, NCHUNK * CH)],
                    tt_v.at[pl.ds(0, NCHUNK * CH)])
    pltpu.sync_copy(pos_hbm.at[pl.ds(0, L)], pos_v)
    pltpu.sync_copy(type_hbm, type_v)
    pltpu.sync_copy(gam_hbm, gb_v.at[0])
    pltpu.sync_copy(bet_hbm, gb_v.at[1])

    ty0 = [type_v[0, pl.ds(f * 16, 16)] for f in range(NF)]
    dty = [type_v[1, pl.ds(f * 16, 16)] - ty0[f] for f in range(NF)]
    gam = [gb_v[0, pl.ds(f * 16, 16)] for f in range(NF)]
    bet = [gb_v[1, pl.ds(f * 16, 16)] for f in range(NF)]

    def add_ty0(t, c):
        for f in range(NF):
            pos_v[t, pl.ds(f * 16, 16)] = pos_v[t, pl.ds(f * 16, 16)] + ty0[f]
        return c
    lax.fori_loop(0, L, add_ty0, 0)

    def gather(c, b):
        pltpu.async_copy(word_hbm.at[ids_v.at[c]], rows[b], gsem[b])

    def wait_gather(b):
        pltpu.make_async_copy(out_hbm.at[pl.ds(0, CH)], rows[b], gsem[b]).wait()

    def store(c, b):
        pltpu.async_copy(rows[b], out_hbm.at[pl.ds((cbase + c) * CH, CH)], ssem[b])

    def wait_store(b):
        pltpu.make_async_copy(rows[b], out_hbm.at[pl.ds(0, CH)], ssem[b]).wait()

    gather(0, 0)
    gather(1, 1)

    def compute_chunk(c, b):
        rv = rows[b]
        pbase = (c & 1) * CH
        toff = c * CH

        def do_grp(g, carry):
            t0 = g * GRP
            ttg = tt_v[pl.ds(toff + t0, 16)]
            for j in range(GRP):
                t = t0 + j
                ttf = ttg[j].astype(jnp.float32)
                x = []
                for f in range(NF):
                    w = rv[t, pl.ds(f * 16, 16)]
                    p = pos_v[pbase + t, pl.ds(f * 16, 16)]
                    x.append(w + (p + ttf * dty[f]))
                acc = (x[0] + x[1]) + (x[2] + x[3])
                acc2 = (x[4] + x[5]) + (x[6] + x[7])
                mu = jnp.sum(acc + acc2) * (1.0 / H)
                sq = [xf * xf for xf in x]
                s1 = (sq[0] + sq[1]) + (sq[2] + sq[3])
                s2 = (sq[4] + sq[5]) + (sq[6] + sq[7])
                var = jnp.sum(s1 + s2) * (1.0 / H) - mu * mu
                v = var + EPS
                i = lax.bitcast_convert_type(v, jnp.int32)
                i = jnp.int32(0x5F3759DF) - lax.shift_right_arithmetic(i, 1)
                y = lax.bitcast_convert_type(i, jnp.float32)
                for _ in range(3):
                    y = y * (1.5 - 0.5 * v * y * y)
                for f in range(NF):
                    g2 = gam[f] * y
                    rv[t, pl.ds(f * 16, 16)] = x[f] * g2 + (bet[f] - mu * g2)
            return carry

        lax.fori_loop(0, CH // GRP, do_grp, 0)

    def do_iter(s2, carry):
        for bb in range(NBUF):
            c = s2 * NBUF + bb

            @pl.when(c + 2 < NCHUNK)
            def _():
                nb = (bb + 2) % NBUF

                @pl.when(c >= 2)
                def _():
                    wait_store(nb)
                gather(c + 2, nb)

            wait_gather(bb)
            compute_chunk(c, bb)
            store(c, bb)
        return carry

    lax.fori_loop(0, NCHUNK // NBUF, do_iter, 0)
    for bb in range(NBUF):
        wait_store(bb)


@jax.jit
def _run(ids2, tt1, word_emb, pos_emb, type_emb, ln_gamma, ln_beta):
    mesh = plsc.VectorSubcoreMesh(core_axis_name="c", subcore_axis_name="s",
                                  num_cores=NC, num_subcores=NS)
    k = pl.kernel(
        _sc_body,
        out_type=jax.ShapeDtypeStruct((B * L, H), jnp.float32),
        mesh=mesh,
        scratch_types=[
            pltpu.VMEM((NCHUNK, CH), jnp.int32),
            pltpu.VMEM((NCHUNK * CH + 16,), jnp.int32),
            pltpu.VMEM((L, H), jnp.float32),
            pltpu.VMEM((2, H), jnp.float32),
            pltpu.VMEM((2, H), jnp.float32),
        ] + [pltpu.VMEM((CH, H), jnp.float32) for _ in range(NBUF)]
          + [pltpu.SemaphoreType.DMA for _ in range(2 * NBUF)],
        compiler_params=pltpu.CompilerParams(needs_layout_passes=False,
                                             use_tc_tiling_on_sc=False),
    )
    return k(ids2, tt1, word_emb, pos_emb, type_emb, ln_gamma, ln_beta)


def kernel(input_ids, token_type_ids, word_emb, pos_emb, type_emb, ln_gamma, ln_beta):
    ids2 = input_ids.astype(jnp.int32).reshape(B * 2, CH)
    tt1 = token_type_ids.astype(jnp.int32).reshape(B * L)
    out = _run(ids2, tt1, word_emb, pos_emb, type_emb, ln_gamma, ln_beta)
    return out.reshape(B, L, H)
